# baseline (speedup 1.0000x reference)
import functools

import jax
import jax.numpy as jnp
from jax import lax
from jax.experimental import pallas as pl
from jax.experimental.pallas import tpu as pltpu
from jax.experimental.pallas import tpu_sc as plsc

_LN_EPS = 1e-5


def _fuse_body(ct_ref, cw_ref, ft_ref, fw_ref, g_ref, b_ref,
               loc_ref, cid_ref, fid_ref, out_ref):
    r = loc_ref.shape[0]
    nc = ct_ref.shape[0]
    nf = ft_ref.shape[0]
    pc = jnp.dot(ct_ref[...], cw_ref[...],
                 preferred_element_type=jnp.float32) * 0.3
    pf = jnp.dot(ft_ref[...], fw_ref[...],
                 preferred_element_type=jnp.float32) * 0.2
    cid = cid_ref[0]
    fid = fid_ref[0]
    oht_c = (cid == lax.broadcasted_iota(jnp.int32, (nc, r), 0)
             ).astype(jnp.float32)
    oht_f = (fid == lax.broadcasted_iota(jnp.int32, (nf, r), 0)
             ).astype(jnp.float32)
    emb_c = lax.dot_general(oht_c, pc, (((0,), (0,)), ((), ())),
                            preferred_element_type=jnp.float32)
    emb_f = lax.dot_general(oht_f, pf, (((0,), (0,)), ((), ())),
                            preferred_element_type=jnp.float32)
    x = loc_ref[...] + emb_c + emb_f
    mean = jnp.mean(x, axis=-1, keepdims=True)
    xc = x - mean
    var = jnp.mean(xc * xc, axis=-1, keepdims=True)
    out_ref[...] = xc * lax.rsqrt(var + _LN_EPS) * g_ref[...] + b_ref[...]


def _build_fused_table(loc_table, ct, cw_t, ft, fw_t, gamma2, beta2,
                       cid3, fid3, block_rows):
    n, d = loc_table.shape
    nb = n // block_rows
    nc = ct.shape[0]
    nf = ft.shape[0]
    return pl.pallas_call(
        _fuse_body,
        grid=(nb,),
        in_specs=[
            pl.BlockSpec((nc, ct.shape[1]), lambda i: (0, 0)),
            pl.BlockSpec((cw_t.shape[0], d), lambda i: (0, 0)),
            pl.BlockSpec((nf, ft.shape[1]), lambda i: (0, 0)),
            pl.BlockSpec((fw_t.shape[0], d), lambda i: (0, 0)),
            pl.BlockSpec((1, d), lambda i: (0, 0)),
            pl.BlockSpec((1, d), lambda i: (0, 0)),
            pl.BlockSpec((block_rows, d), lambda i: (i, 0)),
            pl.BlockSpec((1, 1, block_rows), lambda i: (i, 0, 0)),
            pl.BlockSpec((1, 1, block_rows), lambda i: (i, 0, 0)),
        ],
        out_specs=pl.BlockSpec((block_rows, d), lambda i: (i, 0)),
        out_shape=jax.ShapeDtypeStruct((n, d), jnp.float32),
    )(ct, cw_t, ft, fw_t, gamma2, beta2, loc_table, cid3, fid3)


_NC, _NS, _LANES = 2, 16, 16
_NW = _NC * _NS
_CHUNK = 128


def _make_gather(n_tokens, d):
    per_w = n_tokens // _NW
    n_chunks = per_w // _CHUNK
    mesh = plsc.VectorSubcoreMesh(core_axis_name="c", subcore_axis_name="s")

    @functools.partial(
        pl.kernel,
        out_type=jax.ShapeDtypeStruct((n_tokens, d), jnp.float32),
        mesh=mesh,
        scratch_types=[
            pltpu.VMEM((_CHUNK,), jnp.int32),
            pltpu.VMEM((_CHUNK, d), jnp.float32),
            pltpu.SemaphoreType.DMA,
        ],
        compiler_params=pltpu.CompilerParams(use_tc_tiling_on_sc=True),
    )
    def gather_k(table_hbm, idx_hbm, out_hbm, idx_v, rows_v, sem):
        wid = lax.axis_index("s") * _NC + lax.axis_index("c")
        base0 = wid * per_w

        def body(j, carry):
            base = base0 + j * _CHUNK
            pltpu.sync_copy(idx_hbm.at[pl.ds(base, _CHUNK)], idx_v)
            pltpu.async_copy(table_hbm.at[idx_v], rows_v, sem).wait()
            pltpu.sync_copy(rows_v, out_hbm.at[pl.ds(base, _CHUNK)])
            return carry

        lax.fori_loop(0, n_chunks, body, 0)

    return gather_k


def kernel(loc_ids, loc_table, cluster_table, cluster_proj_w, freq_table,
           freq_proj_w, ln_gamma, ln_beta, loc_to_cluster, loc_freq_bucket):
    b, s = loc_ids.shape
    n, d = loc_table.shape
    n_tokens = b * s

    nc = 64
    nf = 16
    ct = jnp.zeros((nc, cluster_table.shape[1]), jnp.float32
                   ).at[:cluster_table.shape[0]].set(cluster_table)
    ft = jnp.zeros((nf, freq_table.shape[1]), jnp.float32
                   ).at[:freq_table.shape[0]].set(freq_table)
    cw_t = cluster_proj_w.T
    fw_t = freq_proj_w.T
    gamma2 = ln_gamma.reshape(1, d)
    beta2 = ln_beta.reshape(1, d)

    block_rows = 2000
    nb = n // block_rows
    cid3 = loc_to_cluster.reshape(nb, 1, block_rows)
    fid3 = loc_freq_bucket.reshape(nb, 1, block_rows)

    fused = _build_fused_table(loc_table, ct, cw_t, ft, fw_t, gamma2, beta2,
                               cid3, fid3, block_rows)

    flat_ids = loc_ids.reshape(-1).astype(jnp.int32)
    out = _make_gather(n_tokens, d)(fused, flat_ids)
    return out.reshape(b, s, d)


# speedup vs baseline: 1.7049x; 1.7049x over previous
import functools

import jax
import jax.numpy as jnp
from jax import lax
from jax.experimental import pallas as pl
from jax.experimental.pallas import tpu as pltpu
from jax.experimental.pallas import tpu_sc as plsc

_LN_EPS = 1e-5


def _fuse_body(ct_ref, cw_ref, ft_ref, fw_ref, g_ref, b_ref,
               loc_ref, cid_ref, fid_ref, out_ref, pc_ref, pf_ref):
    r = loc_ref.shape[0]
    nc = ct_ref.shape[0]
    nf = ft_ref.shape[0]

    @pl.when(pl.program_id(0) == 0)
    def _():
        pc_ref[...] = jnp.dot(ct_ref[...], cw_ref[...],
                              preferred_element_type=jnp.float32) * 0.3
        pf_ref[...] = jnp.dot(ft_ref[...], fw_ref[...],
                              preferred_element_type=jnp.float32) * 0.2

    pc = pc_ref[...]
    pf = pf_ref[...]
    cid = cid_ref[0]
    fid = fid_ref[0]
    oht_c = (cid == lax.broadcasted_iota(jnp.int32, (nc, r), 0)
             ).astype(jnp.float32)
    oht_f = (fid == lax.broadcasted_iota(jnp.int32, (nf, r), 0)
             ).astype(jnp.float32)
    emb_c = lax.dot_general(oht_c, pc, (((0,), (0,)), ((), ())),
                            preferred_element_type=jnp.float32)
    emb_f = lax.dot_general(oht_f, pf, (((0,), (0,)), ((), ())),
                            preferred_element_type=jnp.float32)
    x = loc_ref[...] + emb_c + emb_f
    mean = jnp.mean(x, axis=-1, keepdims=True)
    xc = x - mean
    var = jnp.mean(xc * xc, axis=-1, keepdims=True)
    out_ref[...] = xc * lax.rsqrt(var + _LN_EPS) * g_ref[...] + b_ref[...]


def _build_fused_table(loc_table, ct, cw_t, ft, fw_t, gamma2, beta2,
                       cid3, fid3, block_rows):
    n, d = loc_table.shape
    nb = n // block_rows
    nc = ct.shape[0]
    nf = ft.shape[0]
    return pl.pallas_call(
        _fuse_body,
        grid=(nb,),
        in_specs=[
            pl.BlockSpec((nc, ct.shape[1]), lambda i: (0, 0)),
            pl.BlockSpec((cw_t.shape[0], d), lambda i: (0, 0)),
            pl.BlockSpec((nf, ft.shape[1]), lambda i: (0, 0)),
            pl.BlockSpec((fw_t.shape[0], d), lambda i: (0, 0)),
            pl.BlockSpec((1, d), lambda i: (0, 0)),
            pl.BlockSpec((1, d), lambda i: (0, 0)),
            pl.BlockSpec((block_rows, d), lambda i: (i, 0)),
            pl.BlockSpec((1, 1, block_rows), lambda i: (i, 0, 0)),
            pl.BlockSpec((1, 1, block_rows), lambda i: (i, 0, 0)),
        ],
        out_specs=pl.BlockSpec((block_rows, d), lambda i: (i, 0)),
        out_shape=jax.ShapeDtypeStruct((n, d), jnp.float32),
        scratch_shapes=[pltpu.VMEM((nc, d), jnp.float32),
                        pltpu.VMEM((nf, d), jnp.float32)],
    )(ct, cw_t, ft, fw_t, gamma2, beta2, loc_table, cid3, fid3)


_NC, _NS, _LANES = 2, 16, 16
_NW = _NC * _NS
_SPAD = 56


def _make_gather(b, s, d):
    bat_per_w = b // _NW
    n_pairs = bat_per_w // 2
    idx_per_w = bat_per_w * _SPAD
    mesh = plsc.VectorSubcoreMesh(core_axis_name="c", subcore_axis_name="s",
                                  num_cores=_NC, num_subcores=_NS)

    @functools.partial(
        pl.kernel,
        out_type=jax.ShapeDtypeStruct((b, s, d), jnp.float32),
        mesh=mesh,
        scratch_types=[
            pltpu.VMEM((idx_per_w,), jnp.int32),
            pltpu.VMEM((2 * _SPAD, d), jnp.float32),
            pltpu.VMEM((2 * _SPAD, d), jnp.float32),
            pltpu.SemaphoreType.DMA,
            pltpu.SemaphoreType.DMA,
        ],
        compiler_params=pltpu.CompilerParams(use_tc_tiling_on_sc=True),
    )
    def gather_k(table_hbm, idx_hbm, out_hbm, idx_v, rows0, rows1, g0, g1):
        wid = lax.axis_index("s") * _NC + lax.axis_index("c")
        bat0 = wid * bat_per_w
        pltpu.sync_copy(idx_hbm.at[pl.ds(wid * idx_per_w, idx_per_w)], idx_v)

        def gather(pair, rows, sem):
            return pltpu.async_copy(
                table_hbm.at[idx_v.at[pl.ds(pair * 2 * _SPAD, 2 * _SPAD)]],
                rows, sem)

        def scatter(pair, rows):
            gb = bat0 + pair * 2
            pltpu.sync_copy(rows.at[pl.ds(0, s)], out_hbm.at[gb])
            pltpu.sync_copy(rows.at[pl.ds(_SPAD, s)], out_hbm.at[gb + 1])

        gather(0, rows0, g0)

        def body(jj, carry):
            gather(2 * jj + 1, rows1, g1)
            pltpu.make_async_copy(table_hbm.at[idx_v.at[pl.ds(0, 2 * _SPAD)]],
                                  rows0, g0).wait()
            scatter(2 * jj, rows0)

            @pl.when(jj < n_pairs // 2 - 1)
            def _():
                gather(2 * jj + 2, rows0, g0)

            pltpu.make_async_copy(table_hbm.at[idx_v.at[pl.ds(0, 2 * _SPAD)]],
                                  rows1, g1).wait()
            scatter(2 * jj + 1, rows1)
            return carry

        lax.fori_loop(0, n_pairs // 2, body, 0)

    return gather_k


def kernel(loc_ids, loc_table, cluster_table, cluster_proj_w, freq_table,
           freq_proj_w, ln_gamma, ln_beta, loc_to_cluster, loc_freq_bucket):
    b, s = loc_ids.shape
    n, d = loc_table.shape
    n_tokens = b * s

    nc = 64
    nf = 16
    ct = jnp.zeros((nc, cluster_table.shape[1]), jnp.float32
                   ).at[:cluster_table.shape[0]].set(cluster_table)
    ft = jnp.zeros((nf, freq_table.shape[1]), jnp.float32
                   ).at[:freq_table.shape[0]].set(freq_table)
    cw_t = cluster_proj_w.T
    fw_t = freq_proj_w.T
    gamma2 = ln_gamma.reshape(1, d)
    beta2 = ln_beta.reshape(1, d)

    block_rows = 2000
    nb = n // block_rows
    cid3 = loc_to_cluster.reshape(nb, 1, block_rows)
    fid3 = loc_freq_bucket.reshape(nb, 1, block_rows)

    fused = _build_fused_table(loc_table, ct, cw_t, ft, fw_t, gamma2, beta2,
                               cid3, fid3, block_rows)

    ids32 = loc_ids.astype(jnp.int32)
    idx_pad = jnp.concatenate(
        [ids32, ids32[:, : _SPAD - s]], axis=1).reshape(-1)
    return _make_gather(b, s, d)(fused, idx_pad)


# speedup vs baseline: 2.7658x; 1.6223x over previous
import functools

import jax
import jax.numpy as jnp
from jax import lax
from jax.experimental import pallas as pl
from jax.experimental.pallas import tpu as pltpu
from jax.experimental.pallas import tpu_sc as plsc

_LN_EPS = 1e-5


def _fuse_body(ct_ref, cw_ref, ft_ref, fw_ref, g_ref, b_ref,
               loc_ref, cid_ref, fid_ref, out_ref, pc_ref, pf_ref):
    r = loc_ref.shape[0]
    nc = ct_ref.shape[0]
    nf = ft_ref.shape[0]

    @pl.when(pl.program_id(0) == 0)
    def _():
        pc_ref[...] = (jnp.dot(ct_ref[...], cw_ref[...],
                               preferred_element_type=jnp.float32)
                       * 0.3).astype(jnp.bfloat16)
        pf_ref[...] = (jnp.dot(ft_ref[...], fw_ref[...],
                               preferred_element_type=jnp.float32)
                       * 0.2).astype(jnp.bfloat16)

    pc = pc_ref[...]
    pf = pf_ref[...]
    cid = cid_ref[0].astype(jnp.int16)
    fid = fid_ref[0].astype(jnp.int16)
    one = jnp.bfloat16(1.0)
    zero = jnp.bfloat16(0.0)
    oht_c = jnp.where(cid == lax.broadcasted_iota(jnp.int16, (nc, r), 0),
                      one, zero)
    oht_f = jnp.where(fid == lax.broadcasted_iota(jnp.int16, (nf, r), 0),
                      one, zero)
    emb_c = lax.dot_general(oht_c, pc, (((0,), (0,)), ((), ())),
                            preferred_element_type=jnp.float32)
    emb_f = lax.dot_general(oht_f, pf, (((0,), (0,)), ((), ())),
                            preferred_element_type=jnp.float32)
    x = loc_ref[...] + emb_c + emb_f
    mean = jnp.mean(x, axis=-1, keepdims=True)
    xc = x - mean
    var = jnp.mean(xc * xc, axis=-1, keepdims=True)
    out_ref[...] = xc * lax.rsqrt(var + _LN_EPS) * g_ref[...] + b_ref[...]


def _build_fused_table(loc_table, ct, cw_t, ft, fw_t, gamma2, beta2,
                       cid3, fid3, block_rows):
    n, d = loc_table.shape
    nb = n // block_rows
    nc = ct.shape[0]
    nf = ft.shape[0]
    return pl.pallas_call(
        _fuse_body,
        grid=(nb,),
        in_specs=[
            pl.BlockSpec((nc, ct.shape[1]), lambda i: (0, 0)),
            pl.BlockSpec((cw_t.shape[0], d), lambda i: (0, 0)),
            pl.BlockSpec((nf, ft.shape[1]), lambda i: (0, 0)),
            pl.BlockSpec((fw_t.shape[0], d), lambda i: (0, 0)),
            pl.BlockSpec((1, d), lambda i: (0, 0)),
            pl.BlockSpec((1, d), lambda i: (0, 0)),
            pl.BlockSpec((block_rows, d), lambda i: (i, 0)),
            pl.BlockSpec((1, 1, block_rows), lambda i: (i, 0, 0)),
            pl.BlockSpec((1, 1, block_rows), lambda i: (i, 0, 0)),
        ],
        out_specs=pl.BlockSpec((block_rows, d), lambda i: (i, 0)),
        out_shape=jax.ShapeDtypeStruct((n, d), jnp.float32),
        scratch_shapes=[pltpu.VMEM((nc, d), jnp.bfloat16),
                        pltpu.VMEM((nf, d), jnp.bfloat16)],
    )(ct, cw_t, ft, fw_t, gamma2, beta2, loc_table, cid3, fid3)


_NC, _NS, _LANES = 2, 16, 16
_NW = _NC * _NS
_CHUNK = 128


def _make_gather(n_tokens, d):
    per_w = n_tokens // _NW
    n_chunks = per_w // _CHUNK
    mesh = plsc.VectorSubcoreMesh(core_axis_name="c", subcore_axis_name="s",
                                  num_cores=_NC, num_subcores=_NS)

    @functools.partial(
        pl.kernel,
        out_type=jax.ShapeDtypeStruct((n_tokens, d), jnp.float32),
        mesh=mesh,
        scratch_types=[
            pltpu.VMEM((per_w,), jnp.int32),
            pltpu.VMEM((_CHUNK, d), jnp.float32),
            pltpu.VMEM((_CHUNK, d), jnp.float32),
            pltpu.SemaphoreType.DMA,
            pltpu.SemaphoreType.DMA,
        ],
        compiler_params=pltpu.CompilerParams(use_tc_tiling_on_sc=True),
    )
    def gather_k(table_hbm, idx_hbm, out_hbm, idx_v, rows0, rows1, g0, g1):
        wid = lax.axis_index("s") * _NC + lax.axis_index("c")
        base0 = wid * per_w
        pltpu.sync_copy(idx_hbm.at[pl.ds(base0, per_w)], idx_v)

        def gather(j, rows, sem):
            return pltpu.async_copy(
                table_hbm.at[idx_v.at[pl.ds(j * _CHUNK, _CHUNK)]], rows, sem)

        def wait(rows, sem):
            pltpu.make_async_copy(table_hbm.at[idx_v.at[pl.ds(0, _CHUNK)]],
                                  rows, sem).wait()

        def scatter(j, rows):
            pltpu.sync_copy(rows, out_hbm.at[pl.ds(base0 + j * _CHUNK,
                                                   _CHUNK)])

        gather(0, rows0, g0)

        def body(jj, carry):
            gather(2 * jj + 1, rows1, g1)
            wait(rows0, g0)
            scatter(2 * jj, rows0)

            @pl.when(jj < n_chunks // 2 - 1)
            def _():
                gather(2 * jj + 2, rows0, g0)

            wait(rows1, g1)
            scatter(2 * jj + 1, rows1)
            return carry

        lax.fori_loop(0, n_chunks // 2, body, 0)

    return gather_k


def kernel(loc_ids, loc_table, cluster_table, cluster_proj_w, freq_table,
           freq_proj_w, ln_gamma, ln_beta, loc_to_cluster, loc_freq_bucket):
    b, s = loc_ids.shape
    n, d = loc_table.shape
    n_tokens = b * s

    ct = cluster_table
    ft = freq_table
    cw_t = cluster_proj_w.T
    fw_t = freq_proj_w.T
    gamma2 = ln_gamma.reshape(1, d)
    beta2 = ln_beta.reshape(1, d)

    block_rows = 4000
    nb = n // block_rows
    cid3 = loc_to_cluster.reshape(nb, 1, block_rows)
    fid3 = loc_freq_bucket.reshape(nb, 1, block_rows)

    fused = _build_fused_table(loc_table, ct, cw_t, ft, fw_t, gamma2, beta2,
                               cid3, fid3, block_rows)

    idx_sm = loc_ids.T.reshape(-1).astype(jnp.int32)
    out_flat = _make_gather(n_tokens, d)(fused, idx_sm)
    return out_flat.reshape(s, b, d).transpose(1, 0, 2)
